# baseline (speedup 1.0000x reference)
import functools

import jax
import jax.numpy as jnp
from jax import lax
from jax.experimental import pallas as pl
from jax.experimental.pallas import tpu as pltpu
from jax.experimental.pallas import tpu_sc as plsc

EMB_DIM = 64
NUM_WORKERS = 32
BCH = 256
NBUF = 2
LANES = 16


def _gather_body(idx_hbm, table_hbm, out_hbm, idx_v, rows_v, trows_v, gsems,
                 osems):
    hist, batch = idx_hbm.shape
    nb = batch // NUM_WORKERS
    nch = nb // BCH
    wid = lax.axis_index("s") * 2 + lax.axis_index("c")
    b0 = wid * nb
    pltpu.sync_copy(idx_hbm.at[:, pl.ds(b0, nb)], idx_v)

    def start_gather(h, c, s):
        pltpu.async_copy(
            table_hbm.at[idx_v.at[h, pl.ds(c * BCH, BCH)]], rows_v.at[s],
            gsems[s])

    def wait_gather(s):
        pltpu.make_async_copy(
            table_hbm.at[idx_v.at[0, pl.ds(0, BCH)]], rows_v.at[s],
            gsems[s]).wait()

    def start_out(h, c, s):
        pltpu.async_copy(
            trows_v.at[s], out_hbm.at[h, :, pl.ds(b0 + c * BCH, BCH)],
            osems[s])

    def wait_out(h, c, s):
        pltpu.make_async_copy(
            trows_v.at[s], out_hbm.at[h, :, pl.ds(b0 + c * BCH, BCH)],
            osems[s]).wait()

    def transpose_chunk(s):
        src = rows_v.at[s]
        dst = trows_v.at[s]

        def dloop(d, carry):
            col = jnp.full((LANES,), d, jnp.int32)
            for g in range(BCH // LANES):
                row = lax.iota(jnp.int32, LANES) + (g * LANES)
                v = plsc.load_gather(src, [row, col])
                dst[d, pl.ds(g * LANES, LANES)] = v
            return carry

        lax.fori_loop(0, EMB_DIM, dloop, 0)

    n = hist * nch

    def hc(k):
        return k // nch, lax.rem(k, nch)

    for s in range(NBUF):
        h, c = hc(s)
        start_gather(h, c, s)

    def body(it, carry):
        k0 = it * NBUF
        for s in range(NBUF):
            h, c = hc(k0 + s)
            wait_gather(s)
            transpose_chunk(s)
            start_out(h, c, s)
        for s in range(NBUF):
            h, c = hc(k0 + s)
            wait_out(h, c, s)
            h2, c2 = hc(k0 + NBUF + s)
            start_gather(h2, c2, s)
        return carry

    lax.fori_loop(0, n // NBUF - 1, body, 0)

    last = n - NBUF
    for s in range(NBUF):
        h, c = hc(last + s)
        wait_gather(s)
        transpose_chunk(s)
        start_out(h, c, s)
    for s in range(NBUF):
        h, c = hc(last + s)
        wait_out(h, c, s)


def kernel(input_, weight):
    batch, hist = input_.shape
    assert batch % (NUM_WORKERS * BCH) == 0
    idx_t = input_.T.astype(jnp.int32)

    mesh = plsc.VectorSubcoreMesh(core_axis_name="c", subcore_axis_name="s")
    run = functools.partial(
        pl.kernel,
        mesh=mesh,
        out_type=jax.ShapeDtypeStruct((hist, EMB_DIM, batch), jnp.float32),
        scratch_types=[
            pltpu.VMEM((hist, batch // NUM_WORKERS), jnp.int32),
            pltpu.VMEM((NBUF, BCH, EMB_DIM), jnp.float32),
            pltpu.VMEM((NBUF, EMB_DIM, BCH), jnp.float32),
            [pltpu.SemaphoreType.DMA] * NBUF,
            [pltpu.SemaphoreType.DMA] * NBUF,
        ],
        compiler_params=pltpu.CompilerParams(
            use_tc_tiling_on_sc=False, needs_layout_passes=False),
    )(_gather_body)
    out = run(idx_t, weight)
    return jnp.transpose(out, (2, 0, 1))


# speedup vs baseline: 1.6773x; 1.6773x over previous
import functools

import jax
import jax.numpy as jnp
from jax import lax
from jax.experimental import pallas as pl
from jax.experimental.pallas import tpu as pltpu
from jax.experimental.pallas import tpu_sc as plsc

EMB_DIM = 64
NUM_WORKERS = 32
CHUNK = 128
NBUF = 8


def _gather_body(table_hbm, idx_hbm, out_hbm, idx_v, rows_v, gsems, osems):
    wid = lax.axis_index("s") * 2 + lax.axis_index("c")
    n_chunks = idx_hbm.shape[0] // NUM_WORKERS
    row_base = wid * n_chunks
    pltpu.sync_copy(idx_hbm.at[pl.ds(row_base, n_chunks)], idx_v)

    def start_gather(j, s):
        pltpu.async_copy(table_hbm.at[idx_v.at[j]], rows_v.at[s], gsems[s])

    def start_out(j, s):
        pltpu.async_copy(
            rows_v.at[s], out_hbm.at[pl.ds((row_base + j) * CHUNK, CHUNK)],
            osems[s])

    def wait_gather(s):
        pltpu.make_async_copy(table_hbm.at[idx_v.at[0]], rows_v.at[s],
                              gsems[s]).wait()

    def wait_out(j, s):
        pltpu.make_async_copy(
            rows_v.at[s], out_hbm.at[pl.ds((row_base + j) * CHUNK, CHUNK)],
            osems[s]).wait()

    for s in range(NBUF):
        start_gather(s, s)

    def body(it, carry):
        jj = it * NBUF
        for s in range(NBUF):
            wait_gather(s)
            start_out(jj + s, s)
        for s in range(NBUF):
            wait_out(jj + s, s)
            start_gather(jj + NBUF + s, s)
        return carry

    lax.fori_loop(0, n_chunks // NBUF - 1, body, 0)

    last = n_chunks - NBUF
    for s in range(NBUF):
        wait_gather(s)
        start_out(last + s, s)
    for s in range(NBUF):
        wait_out(last + s, s)


def kernel(input_, weight):
    batch_shape = input_.shape
    total = input_.size
    assert total % (NUM_WORKERS * CHUNK * NBUF) == 0
    idx2d = input_.reshape(total // CHUNK, CHUNK).astype(jnp.int32)
    n_chunks = idx2d.shape[0] // NUM_WORKERS

    mesh = plsc.VectorSubcoreMesh(core_axis_name="c", subcore_axis_name="s")
    run = functools.partial(
        pl.kernel,
        mesh=mesh,
        out_type=jax.ShapeDtypeStruct((total, EMB_DIM), jnp.float32),
        scratch_types=[
            pltpu.VMEM((n_chunks, CHUNK), jnp.int32),
            pltpu.VMEM((NBUF, CHUNK, EMB_DIM), jnp.float32),
            [pltpu.SemaphoreType.DMA] * NBUF,
            [pltpu.SemaphoreType.DMA] * NBUF,
        ],
        compiler_params=pltpu.CompilerParams(use_tc_tiling_on_sc=False),
    )(_gather_body)
    out = run(weight, idx2d)
    return out.reshape(batch_shape + (EMB_DIM,))
